# baseline (speedup 1.0000x reference)
import functools

import jax
import jax.numpy as jnp
from jax import lax
from jax.experimental import pallas as pl
from jax.experimental.pallas import tpu as pltpu
from jax.experimental.pallas import tpu_sc as plsc

N_NODES = 10000
N_EDGES = 40000
N_GRAPHS = 64
N_PAD = 10240
E_PAD = 40960
NW = 32
CH = E_PAD // NW
KJ = CH // 128
SEG = N_PAD // 16
TN = 1024
TE = 512


def _elu(z):
    return jnp.where(z > 0, z, jnp.exp(jnp.minimum(z, 0.0)) - 1.0)


def _embed_body(x_ref, emb_ref, o_ref):
    x = x_ref[...]
    oh = (lax.broadcasted_iota(jnp.int32, (TN, 8), 1) == x).astype(jnp.float32)
    he = jnp.dot(oh, emb_ref[...], preferred_element_type=jnp.float32)
    o_ref[...] = jnp.concatenate(
        [he, x.astype(jnp.float32), jnp.zeros((TN, 7), jnp.float32)], axis=1)


def _embed(x_col, emb_pad):
    return pl.pallas_call(
        _embed_body,
        grid=(N_PAD // TN,),
        in_specs=[
            pl.BlockSpec((TN, 1), lambda i: (i, 0)),
            pl.BlockSpec((8, 8), lambda i: (0, 0)),
        ],
        out_specs=pl.BlockSpec((TN, 16), lambda i: (i, 0)),
        out_shape=jax.ShapeDtypeStruct((N_PAD, 16), jnp.float32),
    )(x_col, emb_pad)


def _gather_rows(h, src4d, d):
    mesh = plsc.VectorSubcoreMesh(core_axis_name="c", subcore_axis_name="s", num_cores=2, num_subcores=16)

    @functools.partial(
        pl.kernel,
        out_type=jax.ShapeDtypeStruct((NW, KJ, 128, d), jnp.float32),
        mesh=mesh,
        scratch_types=[
            pltpu.VMEM((KJ, 128), jnp.int32),
            pltpu.VMEM((KJ, 128, d), jnp.float32),
            pltpu.SemaphoreType.DMA,
        ],
        compiler_params=pltpu.CompilerParams(use_tc_tiling_on_sc=False),
    )
    def k(h_hbm, src_hbm, out_hbm, idx_v, rows_v, sem):
        c = lax.axis_index("c")
        s = lax.axis_index("s")
        wid = s * 2 + c
        pltpu.sync_copy(src_hbm.at[wid], idx_v)
        descs = [
            pltpu.async_copy(h_hbm.at[idx_v.at[j]], rows_v.at[j], sem)
            for j in range(KJ)
        ]
        for dd in descs:
            dd.wait()
        pltpu.sync_copy(rows_v, out_hbm.at[wid])

    return k(h, src4d).reshape(E_PAD, d)


def _scatter_add(msg, dst4d, zeros_nd, d):
    mesh = plsc.VectorSubcoreMesh(core_axis_name="c", subcore_axis_name="s", num_cores=2, num_subcores=16)
    msg4 = msg.reshape(NW, KJ, 128, d)

    @functools.partial(
        pl.kernel,
        out_type=jax.ShapeDtypeStruct((2, N_PAD, d), jnp.float32),
        mesh=mesh,
        scratch_types=[
            pltpu.VMEM((KJ, 128), jnp.int32),
            pltpu.VMEM((KJ, 128, d), jnp.float32),
            pltpu.VMEM_SHARED((N_PAD, d), jnp.float32),
        ],
        compiler_params=pltpu.CompilerParams(use_tc_tiling_on_sc=False),
    )
    def k(msg_hbm, dst_hbm, z_hbm, out_hbm, idx_v, rows_v, shared):
        c = lax.axis_index("c")
        s = lax.axis_index("s")
        wid = s * 2 + c
        pltpu.sync_copy(z_hbm.at[pl.ds(s * SEG, SEG)],
                        shared.at[pl.ds(s * SEG, SEG)])
        pltpu.sync_copy(dst_hbm.at[wid], idx_v)
        pltpu.sync_copy(msg_hbm.at[wid], rows_v)
        plsc.subcore_barrier()
        for j in range(KJ):
            pltpu.sync_copy(rows_v.at[j], shared.at[idx_v.at[j]], add=True)
        plsc.subcore_barrier()
        pltpu.sync_copy(shared.at[pl.ds(s * SEG, SEG)],
                        out_hbm.at[c].at[pl.ds(s * SEG, SEG)])

    return k(msg4, dst4d, zeros_nd)


_TLHS = (((0,), (0,)), ((), ()))


def _msg_call(ea, hsrc, l1w, l1bT, l2w, l2bm, mi, mo, mi_p):
    def body(ea_ref, hs_ref, l1w_ref, l1b_ref, l2w_ref, l2b_ref, o_ref):
        ea_t = ea_ref[...].T
        hid_t = jnp.maximum(
            lax.dot_general(l1w_ref[...], ea_t, _TLHS,
                            preferred_element_type=jnp.float32)
            + l1b_ref[...], 0.0)
        wt_t = lax.dot_general(l2w_ref[...], hid_t, _TLHS,
                               preferred_element_type=jnp.float32)
        hs_t = hs_ref[...].T
        acc = lax.dot_general(l2b_ref[...], hs_t, _TLHS,
                              preferred_element_type=jnp.float32)
        for i in range(mi):
            acc = acc + hs_t[i:i + 1, :] * wt_t[i * mo:(i + 1) * mo, :]
        o_ref[...] = acc.T

    return pl.pallas_call(
        body,
        grid=(E_PAD // TE,),
        in_specs=[
            pl.BlockSpec((TE, 16), lambda i: (i, 0)),
            pl.BlockSpec((TE, mi_p), lambda i: (i, 0)),
            pl.BlockSpec((16, 128), lambda i: (0, 0)),
            pl.BlockSpec((128, 1), lambda i: (0, 0)),
            pl.BlockSpec((128, mi * mo), lambda i: (0, 0)),
            pl.BlockSpec((mi_p, mo), lambda i: (0, 0)),
        ],
        out_specs=pl.BlockSpec((TE, mo), lambda i: (i, 0)),
        out_shape=jax.ShapeDtypeStruct((E_PAD, mo), jnp.float32),
    )(ea, hsrc, l1w, l1bT, l2w, l2bm)


def _update_call(h, agg0, agg1, root, bias, mi_p, mo):
    def body(h_ref, a0_ref, a1_ref, r_ref, b_ref, o_ref):
        z = (a0_ref[...] + a1_ref[...] +
             jnp.dot(h_ref[...], r_ref[...],
                     preferred_element_type=jnp.float32) + b_ref[...])
        o_ref[...] = _elu(z)

    return pl.pallas_call(
        body,
        grid=(N_PAD // TN,),
        in_specs=[
            pl.BlockSpec((TN, mi_p), lambda i: (i, 0)),
            pl.BlockSpec((TN, mo), lambda i: (i, 0)),
            pl.BlockSpec((TN, mo), lambda i: (i, 0)),
            pl.BlockSpec((mi_p, mo), lambda i: (0, 0)),
            pl.BlockSpec((1, mo), lambda i: (0, 0)),
        ],
        out_specs=pl.BlockSpec((TN, mo), lambda i: (i, 0)),
        out_shape=jax.ShapeDtypeStruct((N_PAD, mo), jnp.float32),
    )(h, agg0, agg1, root, bias)


def _update_readout_call(h, agg0, agg1, root, bias, batch_row,
                         fc1w, fc1b, fc2w, fc2b, fc3w, fc3b, mi_p, mo):
    nsteps = N_PAD // TN

    def body(h_ref, a0_ref, a1_ref, r_ref, b_ref, bt_ref, w1_ref, b1_ref,
             w2_ref, b2_ref, w3_ref, b3_ref, o_ref, acc_ref):
        step = pl.program_id(0)

        @pl.when(step == 0)
        def _init():
            acc_ref[...] = jnp.zeros((128, mo + 1), jnp.float32)

        z = (a0_ref[...] + a1_ref[...] +
             jnp.dot(h_ref[...], r_ref[...],
                     preferred_element_type=jnp.float32) + b_ref[...])
        h3 = jnp.concatenate([_elu(z), jnp.ones((TN, 1), jnp.float32)],
                             axis=1)
        oh = (lax.broadcasted_iota(jnp.int32, (128, TN), 0)
              == bt_ref[...]).astype(jnp.float32)
        acc_ref[...] += jnp.dot(oh, h3, preferred_element_type=jnp.float32)

        @pl.when(step == nsteps - 1)
        def _fin():
            v = acc_ref[...]
            hg = v[:64, :mo] / jnp.maximum(v[:64, mo:mo + 1], 1.0)
            h1 = _elu(jnp.dot(hg, w1_ref[...],
                              preferred_element_type=jnp.float32) + b1_ref[...])
            h2 = _elu(jnp.dot(h1, w2_ref[...],
                              preferred_element_type=jnp.float32) + b2_ref[...])
            o_ref[...] = jnp.dot(h2, w3_ref[...],
                                 preferred_element_type=jnp.float32) + b3_ref[...]

    return pl.pallas_call(
        body,
        grid=(nsteps,),
        in_specs=[
            pl.BlockSpec((TN, mi_p), lambda i: (i, 0)),
            pl.BlockSpec((TN, mo), lambda i: (i, 0)),
            pl.BlockSpec((TN, mo), lambda i: (i, 0)),
            pl.BlockSpec((mi_p, mo), lambda i: (0, 0)),
            pl.BlockSpec((1, mo), lambda i: (0, 0)),
            pl.BlockSpec((1, TN), lambda i: (0, i)),
            pl.BlockSpec((64, 32), lambda i: (0, 0)),
            pl.BlockSpec((1, 32), lambda i: (0, 0)),
            pl.BlockSpec((32, 16), lambda i: (0, 0)),
            pl.BlockSpec((1, 16), lambda i: (0, 0)),
            pl.BlockSpec((16, 1), lambda i: (0, 0)),
            pl.BlockSpec((1, 1), lambda i: (0, 0)),
        ],
        out_specs=pl.BlockSpec((64, 1), lambda i: (0, 0)),
        out_shape=jax.ShapeDtypeStruct((64, 1), jnp.float32),
        scratch_shapes=[pltpu.VMEM((128, mo + 1), jnp.float32)],
    )(h, agg0, agg1, root, bias, batch_row,
      fc1w, fc1b, fc2w, fc2b, fc3w, fc3b)


DIMS_K = [(9, 16, 32), (32, 32, 64), (64, 64, 64)]


def kernel(x, edge_index, edge_attr, batch, emb_table,
           c0_l1_w, c0_l1_b, c0_l2_w, c0_l2_b, c0_root, c0_bias,
           c1_l1_w, c1_l1_b, c1_l2_w, c1_l2_b, c1_root, c1_bias,
           c2_l1_w, c2_l1_b, c2_l2_w, c2_l2_b, c2_root, c2_bias,
           fc1_w, fc1_b, fc2_w, fc2_b, fc3_w, fc3_b):
    f32 = jnp.float32
    i32 = jnp.int32

    x_col = jnp.pad(x.astype(i32), (0, N_PAD - N_NODES)).reshape(N_PAD, 1)
    src4d = jnp.pad(edge_index[0].astype(i32),
                    (0, E_PAD - N_EDGES)).reshape(NW, KJ, 128)
    dst4d = jnp.pad(edge_index[1].astype(i32), (0, E_PAD - N_EDGES),
                    constant_values=N_NODES).reshape(NW, KJ, 128)
    ea = jnp.pad(edge_attr.astype(f32), ((0, E_PAD - N_EDGES), (0, 0)))
    batch_row = jnp.pad(batch.astype(i32), (0, N_PAD - N_NODES),
                        constant_values=127).reshape(1, N_PAD)
    emb_pad = jnp.pad(emb_table.astype(f32), ((0, 3), (0, 0)))

    layers = [
        (c0_l1_w, c0_l1_b, c0_l2_w, c0_l2_b, c0_root, c0_bias),
        (c1_l1_w, c1_l1_b, c1_l2_w, c1_l2_b, c1_root, c1_bias),
        (c2_l1_w, c2_l1_b, c2_l2_w, c2_l2_b, c2_root, c2_bias),
    ]

    h = _embed(x_col, emb_pad)

    for li, (mi, mi_p, mo) in enumerate(DIMS_K):
        l1w, l1b, l2w, l2b, root, bias = layers[li]
        l2bm = jnp.pad(l2b.reshape(mi, mo), ((0, mi_p - mi), (0, 0)))
        root_p = jnp.pad(root, ((0, mi_p - mi), (0, 0)))
        bias2 = bias.reshape(1, mo)
        zeros_nd = jnp.zeros((N_PAD, mo), f32)

        hsrc = _gather_rows(h, src4d, mi_p)
        msg = _msg_call(ea, hsrc, l1w, l1b.reshape(128, 1), l2w,
                        l2bm, mi, mo, mi_p)
        aggp = _scatter_add(msg, dst4d, zeros_nd, mo)
        if li < 2:
            h = _update_call(h, aggp[0], aggp[1], root_p, bias2, mi_p, mo)
        else:
            return _update_readout_call(
                h, aggp[0], aggp[1], root_p, bias2, batch_row,
                fc1_w, fc1_b.reshape(1, 32),
                fc2_w, fc2_b.reshape(1, 16),
                fc3_w, fc3_b.reshape(1, 1), mi_p, mo)


# speedup vs baseline: 1.0422x; 1.0422x over previous
import functools

import jax
import jax.numpy as jnp
from jax import lax
from jax.experimental import pallas as pl
from jax.experimental.pallas import tpu as pltpu
from jax.experimental.pallas import tpu_sc as plsc

N_NODES = 10000
N_EDGES = 40000
N_GRAPHS = 64
N_PAD = 10240
E_PAD = 40960
NW = 32
CH = E_PAD // NW
KJ = CH // 128
SEG = N_PAD // 16
TN = 1024
TE = 512


def _elu(z):
    return jnp.where(z > 0, z, jnp.exp(jnp.minimum(z, 0.0)) - 1.0)


def _embed_body(x_ref, emb_ref, o_ref):
    x = x_ref[...]
    oh = (lax.broadcasted_iota(jnp.int32, (TN, 8), 1) == x).astype(jnp.float32)
    he = jnp.dot(oh, emb_ref[...], preferred_element_type=jnp.float32)
    o_ref[...] = jnp.concatenate(
        [he, x.astype(jnp.float32), jnp.zeros((TN, 7), jnp.float32)], axis=1)


def _embed(x_col, emb_pad):
    return pl.pallas_call(
        _embed_body,
        grid=(N_PAD // TN,),
        in_specs=[
            pl.BlockSpec((TN, 1), lambda i: (i, 0)),
            pl.BlockSpec((8, 8), lambda i: (0, 0)),
        ],
        out_specs=pl.BlockSpec((TN, 16), lambda i: (i, 0)),
        out_shape=jax.ShapeDtypeStruct((N_PAD, 16), jnp.float32),
    )(x_col, emb_pad)


def _gather_rows(h, src4d, d):
    mesh = plsc.VectorSubcoreMesh(core_axis_name="c", subcore_axis_name="s", num_cores=2, num_subcores=16)

    @functools.partial(
        pl.kernel,
        out_type=jax.ShapeDtypeStruct((NW, KJ, 128, d), jnp.float32),
        mesh=mesh,
        scratch_types=[
            pltpu.VMEM((KJ, 128), jnp.int32),
            pltpu.VMEM((KJ, 128, d), jnp.float32),
            pltpu.SemaphoreType.DMA,
        ],
        compiler_params=pltpu.CompilerParams(use_tc_tiling_on_sc=False),
    )
    def k(h_hbm, src_hbm, out_hbm, idx_v, rows_v, sem):
        c = lax.axis_index("c")
        s = lax.axis_index("s")
        wid = s * 2 + c
        pltpu.sync_copy(src_hbm.at[wid], idx_v)
        descs = [
            pltpu.async_copy(h_hbm.at[idx_v.at[j]], rows_v.at[j], sem)
            for j in range(KJ)
        ]
        for dd in descs:
            dd.wait()
        pltpu.sync_copy(rows_v, out_hbm.at[wid])

    return k(h, src4d).reshape(E_PAD, d)


def _scatter_add(msg, dst4d, zeros_nd, d):
    mesh = plsc.VectorSubcoreMesh(core_axis_name="c", subcore_axis_name="s", num_cores=2, num_subcores=16)
    msg4 = msg.reshape(NW, KJ, 128, d)

    @functools.partial(
        pl.kernel,
        out_type=jax.ShapeDtypeStruct((2, N_PAD, d), jnp.float32),
        mesh=mesh,
        scratch_types=[
            pltpu.VMEM((KJ, 128), jnp.int32),
            pltpu.VMEM((KJ, 128, d), jnp.float32),
            pltpu.VMEM_SHARED((N_PAD, d), jnp.float32),
        ],
        compiler_params=pltpu.CompilerParams(use_tc_tiling_on_sc=False),
    )
    def k(msg_hbm, dst_hbm, z_hbm, out_hbm, idx_v, rows_v, shared):
        c = lax.axis_index("c")
        s = lax.axis_index("s")
        wid = s * 2 + c
        pltpu.sync_copy(z_hbm.at[pl.ds(s * SEG, SEG)],
                        shared.at[pl.ds(s * SEG, SEG)])
        pltpu.sync_copy(dst_hbm.at[wid], idx_v)
        pltpu.sync_copy(msg_hbm.at[wid], rows_v)
        plsc.subcore_barrier()
        for j in range(KJ):
            pltpu.sync_copy(rows_v.at[j], shared.at[idx_v.at[j]], add=True)
        plsc.subcore_barrier()
        pltpu.sync_copy(shared.at[pl.ds(s * SEG, SEG)],
                        out_hbm.at[c].at[pl.ds(s * SEG, SEG)])

    return k(msg4, dst4d, zeros_nd)


def _msg_call(ea, hsrc, l1wT, l1bT, l2wT, l2bmT, mi, mo, mi_p):
    def body(ea_ref, hs_ref, l1w_ref, l1b_ref, l2w_ref, l2b_ref, o_ref):
        ea_t = ea_ref[...].T
        hid_t = jnp.maximum(
            jnp.dot(l1w_ref[...], ea_t,
                    preferred_element_type=jnp.float32) + l1b_ref[...], 0.0)
        wt_t = jnp.dot(l2w_ref[...], hid_t,
                       preferred_element_type=jnp.float32)
        hs_t = hs_ref[...].T
        acc = jnp.dot(l2b_ref[...], hs_t,
                      preferred_element_type=jnp.float32)
        for i in range(mi):
            acc = acc + hs_t[i:i + 1, :] * wt_t[i * mo:(i + 1) * mo, :]
        o_ref[...] = acc.T

    return pl.pallas_call(
        body,
        grid=(E_PAD // TE,),
        in_specs=[
            pl.BlockSpec((TE, 16), lambda i: (i, 0)),
            pl.BlockSpec((TE, mi_p), lambda i: (i, 0)),
            pl.BlockSpec((128, 16), lambda i: (0, 0)),
            pl.BlockSpec((128, 1), lambda i: (0, 0)),
            pl.BlockSpec((mi * mo, 128), lambda i: (0, 0)),
            pl.BlockSpec((mo, mi_p), lambda i: (0, 0)),
        ],
        out_specs=pl.BlockSpec((TE, mo), lambda i: (i, 0)),
        out_shape=jax.ShapeDtypeStruct((E_PAD, mo), jnp.float32),
    )(ea, hsrc, l1wT, l1bT, l2wT, l2bmT)


def _update_call(h, agg0, agg1, root, bias, mi_p, mo):
    def body(h_ref, a0_ref, a1_ref, r_ref, b_ref, o_ref):
        z = (a0_ref[...] + a1_ref[...] +
             jnp.dot(h_ref[...], r_ref[...],
                     preferred_element_type=jnp.float32) + b_ref[...])
        o_ref[...] = _elu(z)

    return pl.pallas_call(
        body,
        grid=(N_PAD // TN,),
        in_specs=[
            pl.BlockSpec((TN, mi_p), lambda i: (i, 0)),
            pl.BlockSpec((TN, mo), lambda i: (i, 0)),
            pl.BlockSpec((TN, mo), lambda i: (i, 0)),
            pl.BlockSpec((mi_p, mo), lambda i: (0, 0)),
            pl.BlockSpec((1, mo), lambda i: (0, 0)),
        ],
        out_specs=pl.BlockSpec((TN, mo), lambda i: (i, 0)),
        out_shape=jax.ShapeDtypeStruct((N_PAD, mo), jnp.float32),
    )(h, agg0, agg1, root, bias)


def _update_readout_call(h, agg0, agg1, root, bias, batch_row,
                         fc1w, fc1b, fc2w, fc2b, fc3w, fc3b, mi_p, mo):
    nsteps = N_PAD // TN

    def body(h_ref, a0_ref, a1_ref, r_ref, b_ref, bt_ref, w1_ref, b1_ref,
             w2_ref, b2_ref, w3_ref, b3_ref, o_ref, acc_ref):
        step = pl.program_id(0)

        @pl.when(step == 0)
        def _init():
            acc_ref[...] = jnp.zeros((128, mo + 1), jnp.float32)

        z = (a0_ref[...] + a1_ref[...] +
             jnp.dot(h_ref[...], r_ref[...],
                     preferred_element_type=jnp.float32) + b_ref[...])
        h3 = jnp.concatenate([_elu(z), jnp.ones((TN, 1), jnp.float32)],
                             axis=1)
        oh = (lax.broadcasted_iota(jnp.int32, (128, TN), 0)
              == bt_ref[...]).astype(jnp.float32)
        acc_ref[...] += jnp.dot(oh, h3, preferred_element_type=jnp.float32)

        @pl.when(step == nsteps - 1)
        def _fin():
            v = acc_ref[...]
            hg = v[:64, :mo] / jnp.maximum(v[:64, mo:mo + 1], 1.0)
            h1 = _elu(jnp.dot(hg, w1_ref[...],
                              preferred_element_type=jnp.float32) + b1_ref[...])
            h2 = _elu(jnp.dot(h1, w2_ref[...],
                              preferred_element_type=jnp.float32) + b2_ref[...])
            o_ref[...] = jnp.dot(h2, w3_ref[...],
                                 preferred_element_type=jnp.float32) + b3_ref[...]

    return pl.pallas_call(
        body,
        grid=(nsteps,),
        in_specs=[
            pl.BlockSpec((TN, mi_p), lambda i: (i, 0)),
            pl.BlockSpec((TN, mo), lambda i: (i, 0)),
            pl.BlockSpec((TN, mo), lambda i: (i, 0)),
            pl.BlockSpec((mi_p, mo), lambda i: (0, 0)),
            pl.BlockSpec((1, mo), lambda i: (0, 0)),
            pl.BlockSpec((1, TN), lambda i: (0, i)),
            pl.BlockSpec((64, 32), lambda i: (0, 0)),
            pl.BlockSpec((1, 32), lambda i: (0, 0)),
            pl.BlockSpec((32, 16), lambda i: (0, 0)),
            pl.BlockSpec((1, 16), lambda i: (0, 0)),
            pl.BlockSpec((16, 1), lambda i: (0, 0)),
            pl.BlockSpec((1, 1), lambda i: (0, 0)),
        ],
        out_specs=pl.BlockSpec((64, 1), lambda i: (0, 0)),
        out_shape=jax.ShapeDtypeStruct((64, 1), jnp.float32),
        scratch_shapes=[pltpu.VMEM((128, mo + 1), jnp.float32)],
    )(h, agg0, agg1, root, bias, batch_row,
      fc1w, fc1b, fc2w, fc2b, fc3w, fc3b)


DIMS_K = [(9, 16, 32), (32, 32, 64), (64, 64, 64)]


def kernel(x, edge_index, edge_attr, batch, emb_table,
           c0_l1_w, c0_l1_b, c0_l2_w, c0_l2_b, c0_root, c0_bias,
           c1_l1_w, c1_l1_b, c1_l2_w, c1_l2_b, c1_root, c1_bias,
           c2_l1_w, c2_l1_b, c2_l2_w, c2_l2_b, c2_root, c2_bias,
           fc1_w, fc1_b, fc2_w, fc2_b, fc3_w, fc3_b):
    f32 = jnp.float32
    i32 = jnp.int32

    x_col = jnp.pad(x.astype(i32), (0, N_PAD - N_NODES)).reshape(N_PAD, 1)
    src4d = jnp.pad(edge_index[0].astype(i32),
                    (0, E_PAD - N_EDGES)).reshape(NW, KJ, 128)
    dst4d = jnp.pad(edge_index[1].astype(i32), (0, E_PAD - N_EDGES),
                    constant_values=N_NODES).reshape(NW, KJ, 128)
    ea = jnp.pad(edge_attr.astype(f32), ((0, E_PAD - N_EDGES), (0, 0)))
    batch_row = jnp.pad(batch.astype(i32), (0, N_PAD - N_NODES),
                        constant_values=127).reshape(1, N_PAD)
    emb_pad = jnp.pad(emb_table.astype(f32), ((0, 3), (0, 0)))

    layers = [
        (c0_l1_w, c0_l1_b, c0_l2_w, c0_l2_b, c0_root, c0_bias),
        (c1_l1_w, c1_l1_b, c1_l2_w, c1_l2_b, c1_root, c1_bias),
        (c2_l1_w, c2_l1_b, c2_l2_w, c2_l2_b, c2_root, c2_bias),
    ]

    h = _embed(x_col, emb_pad)

    for li, (mi, mi_p, mo) in enumerate(DIMS_K):
        l1w, l1b, l2w, l2b, root, bias = layers[li]
        l2bm = jnp.pad(l2b.reshape(mi, mo), ((0, mi_p - mi), (0, 0)))
        root_p = jnp.pad(root, ((0, mi_p - mi), (0, 0)))
        bias2 = bias.reshape(1, mo)
        zeros_nd = jnp.zeros((N_PAD, mo), f32)

        hsrc = _gather_rows(h, src4d, mi_p)
        msg = _msg_call(ea, hsrc, l1w.T, l1b.reshape(128, 1), l2w.T,
                        l2bm.T, mi, mo, mi_p)
        aggp = _scatter_add(msg, dst4d, zeros_nd, mo)
        if li < 2:
            h = _update_call(h, aggp[0], aggp[1], root_p, bias2, mi_p, mo)
        else:
            return _update_readout_call(
                h, aggp[0], aggp[1], root_p, bias2, batch_row,
                fc1_w, fc1_b.reshape(1, 32),
                fc2_w, fc2_b.reshape(1, 16),
                fc3_w, fc3_b.reshape(1, 1), mi_p, mo)


# speedup vs baseline: 1.1004x; 1.0559x over previous
import functools

import jax
import jax.numpy as jnp
from jax import lax
from jax.experimental import pallas as pl
from jax.experimental.pallas import tpu as pltpu
from jax.experimental.pallas import tpu_sc as plsc

N_NODES = 10000
N_EDGES = 40000
N_GRAPHS = 64
N_PAD = 10240
E_PAD = 40960
NW = 32
CH = E_PAD // NW
KJ = CH // 128
SEG = N_PAD // 16
TN = 1024
TE = 1024


def _elu(z):
    return jnp.where(z > 0, z, jnp.exp(jnp.minimum(z, 0.0)) - 1.0)


def _embed_body(x_ref, emb_ref, o_ref):
    x = x_ref[...]
    oh = (lax.broadcasted_iota(jnp.int32, (TN, 8), 1) == x).astype(jnp.float32)
    he = jnp.dot(oh, emb_ref[...], preferred_element_type=jnp.float32)
    o_ref[...] = jnp.concatenate(
        [he, x.astype(jnp.float32), jnp.zeros((TN, 7), jnp.float32)], axis=1)


def _embed(x_col, emb_pad):
    return pl.pallas_call(
        _embed_body,
        grid=(N_PAD // TN,),
        in_specs=[
            pl.BlockSpec((TN, 1), lambda i: (i, 0)),
            pl.BlockSpec((8, 8), lambda i: (0, 0)),
        ],
        out_specs=pl.BlockSpec((TN, 16), lambda i: (i, 0)),
        out_shape=jax.ShapeDtypeStruct((N_PAD, 16), jnp.float32),
    )(x_col, emb_pad)


def _gather_rows(h, src4d, d):
    mesh = plsc.VectorSubcoreMesh(core_axis_name="c", subcore_axis_name="s", num_cores=2, num_subcores=16)

    @functools.partial(
        pl.kernel,
        out_type=jax.ShapeDtypeStruct((NW, KJ, 128, d), jnp.float32),
        mesh=mesh,
        scratch_types=[
            pltpu.VMEM((KJ, 128), jnp.int32),
            pltpu.VMEM((KJ, 128, d), jnp.float32),
            pltpu.SemaphoreType.DMA,
        ],
        compiler_params=pltpu.CompilerParams(use_tc_tiling_on_sc=False),
    )
    def k(h_hbm, src_hbm, out_hbm, idx_v, rows_v, sem):
        c = lax.axis_index("c")
        s = lax.axis_index("s")
        wid = s * 2 + c
        pltpu.sync_copy(src_hbm.at[wid], idx_v)
        descs = [
            pltpu.async_copy(h_hbm.at[idx_v.at[j]], rows_v.at[j], sem)
            for j in range(KJ)
        ]
        for dd in descs:
            dd.wait()
        pltpu.sync_copy(rows_v, out_hbm.at[wid])

    return k(h, src4d).reshape(E_PAD, d)


def _scatter_add(msg, dst4d, zeros_nd, d):
    mesh = plsc.VectorSubcoreMesh(core_axis_name="c", subcore_axis_name="s", num_cores=2, num_subcores=16)
    msg4 = msg.reshape(NW, KJ, 128, d)

    @functools.partial(
        pl.kernel,
        out_type=jax.ShapeDtypeStruct((2, N_PAD, d), jnp.float32),
        mesh=mesh,
        scratch_types=[
            pltpu.VMEM((KJ, 128), jnp.int32),
            pltpu.VMEM((KJ, 128, d), jnp.float32),
            pltpu.VMEM_SHARED((N_PAD, d), jnp.float32),
        ],
        compiler_params=pltpu.CompilerParams(use_tc_tiling_on_sc=False),
    )
    def k(msg_hbm, dst_hbm, z_hbm, out_hbm, idx_v, rows_v, shared):
        c = lax.axis_index("c")
        s = lax.axis_index("s")
        wid = s * 2 + c
        pltpu.sync_copy(z_hbm.at[pl.ds(s * SEG, SEG)],
                        shared.at[pl.ds(s * SEG, SEG)])
        pltpu.sync_copy(dst_hbm.at[wid], idx_v)
        pltpu.sync_copy(msg_hbm.at[wid], rows_v)
        plsc.subcore_barrier()
        for j in range(KJ):
            pltpu.sync_copy(rows_v.at[j], shared.at[idx_v.at[j]], add=True)
        plsc.subcore_barrier()
        pltpu.sync_copy(shared.at[pl.ds(s * SEG, SEG)],
                        out_hbm.at[c].at[pl.ds(s * SEG, SEG)])

    return k(msg4, dst4d, zeros_nd)


def _msg_call(ea, hsrc, l1wT, l1bT, l2wT, l2bmT, mi, mo, mi_p):
    def body(ea_ref, hs_ref, l1w_ref, l1b_ref, l2w_ref, l2b_ref, o_ref):
        ea_t = ea_ref[...].T
        hid_t = jnp.maximum(
            jnp.dot(l1w_ref[...], ea_t,
                    preferred_element_type=jnp.float32) + l1b_ref[...], 0.0)
        wt_t = jnp.dot(l2w_ref[...], hid_t,
                       preferred_element_type=jnp.float32)
        hs_t = hs_ref[...].T
        acc = jnp.dot(l2b_ref[...], hs_t,
                      preferred_element_type=jnp.float32)
        for i in range(mi):
            acc = acc + hs_t[i:i + 1, :] * wt_t[i * mo:(i + 1) * mo, :]
        o_ref[...] = acc.T

    return pl.pallas_call(
        body,
        grid=(E_PAD // TE,),
        in_specs=[
            pl.BlockSpec((TE, 16), lambda i: (i, 0)),
            pl.BlockSpec((TE, mi_p), lambda i: (i, 0)),
            pl.BlockSpec((128, 16), lambda i: (0, 0)),
            pl.BlockSpec((128, 1), lambda i: (0, 0)),
            pl.BlockSpec((mi * mo, 128), lambda i: (0, 0)),
            pl.BlockSpec((mo, mi_p), lambda i: (0, 0)),
        ],
        out_specs=pl.BlockSpec((TE, mo), lambda i: (i, 0)),
        out_shape=jax.ShapeDtypeStruct((E_PAD, mo), jnp.float32),
    )(ea, hsrc, l1wT, l1bT, l2wT, l2bmT)


def _update_call(h, agg0, agg1, root, bias, mi_p, mo):
    def body(h_ref, a0_ref, a1_ref, r_ref, b_ref, o_ref):
        z = (a0_ref[...] + a1_ref[...] +
             jnp.dot(h_ref[...], r_ref[...],
                     preferred_element_type=jnp.float32) + b_ref[...])
        o_ref[...] = _elu(z)

    return pl.pallas_call(
        body,
        grid=(N_PAD // TN,),
        in_specs=[
            pl.BlockSpec((TN, mi_p), lambda i: (i, 0)),
            pl.BlockSpec((TN, mo), lambda i: (i, 0)),
            pl.BlockSpec((TN, mo), lambda i: (i, 0)),
            pl.BlockSpec((mi_p, mo), lambda i: (0, 0)),
            pl.BlockSpec((1, mo), lambda i: (0, 0)),
        ],
        out_specs=pl.BlockSpec((TN, mo), lambda i: (i, 0)),
        out_shape=jax.ShapeDtypeStruct((N_PAD, mo), jnp.float32),
    )(h, agg0, agg1, root, bias)


def _update_readout_call(h, agg0, agg1, root, bias, batch_row,
                         fc1w, fc1b, fc2w, fc2b, fc3w, fc3b, mi_p, mo):
    nsteps = N_PAD // TN

    def body(h_ref, a0_ref, a1_ref, r_ref, b_ref, bt_ref, w1_ref, b1_ref,
             w2_ref, b2_ref, w3_ref, b3_ref, o_ref, acc_ref):
        step = pl.program_id(0)

        @pl.when(step == 0)
        def _init():
            acc_ref[...] = jnp.zeros((128, mo + 1), jnp.float32)

        z = (a0_ref[...] + a1_ref[...] +
             jnp.dot(h_ref[...], r_ref[...],
                     preferred_element_type=jnp.float32) + b_ref[...])
        h3 = jnp.concatenate([_elu(z), jnp.ones((TN, 1), jnp.float32)],
                             axis=1)
        oh = (lax.broadcasted_iota(jnp.int32, (128, TN), 0)
              == bt_ref[...]).astype(jnp.float32)
        acc_ref[...] += jnp.dot(oh, h3, preferred_element_type=jnp.float32)

        @pl.when(step == nsteps - 1)
        def _fin():
            v = acc_ref[...]
            hg = v[:64, :mo] / jnp.maximum(v[:64, mo:mo + 1], 1.0)
            h1 = _elu(jnp.dot(hg, w1_ref[...],
                              preferred_element_type=jnp.float32) + b1_ref[...])
            h2 = _elu(jnp.dot(h1, w2_ref[...],
                              preferred_element_type=jnp.float32) + b2_ref[...])
            o_ref[...] = jnp.dot(h2, w3_ref[...],
                                 preferred_element_type=jnp.float32) + b3_ref[...]

    return pl.pallas_call(
        body,
        grid=(nsteps,),
        in_specs=[
            pl.BlockSpec((TN, mi_p), lambda i: (i, 0)),
            pl.BlockSpec((TN, mo), lambda i: (i, 0)),
            pl.BlockSpec((TN, mo), lambda i: (i, 0)),
            pl.BlockSpec((mi_p, mo), lambda i: (0, 0)),
            pl.BlockSpec((1, mo), lambda i: (0, 0)),
            pl.BlockSpec((1, TN), lambda i: (0, i)),
            pl.BlockSpec((64, 32), lambda i: (0, 0)),
            pl.BlockSpec((1, 32), lambda i: (0, 0)),
            pl.BlockSpec((32, 16), lambda i: (0, 0)),
            pl.BlockSpec((1, 16), lambda i: (0, 0)),
            pl.BlockSpec((16, 1), lambda i: (0, 0)),
            pl.BlockSpec((1, 1), lambda i: (0, 0)),
        ],
        out_specs=pl.BlockSpec((64, 1), lambda i: (0, 0)),
        out_shape=jax.ShapeDtypeStruct((64, 1), jnp.float32),
        scratch_shapes=[pltpu.VMEM((128, mo + 1), jnp.float32)],
    )(h, agg0, agg1, root, bias, batch_row,
      fc1w, fc1b, fc2w, fc2b, fc3w, fc3b)


DIMS_K = [(9, 16, 32), (32, 32, 64), (64, 64, 64)]


def kernel(x, edge_index, edge_attr, batch, emb_table,
           c0_l1_w, c0_l1_b, c0_l2_w, c0_l2_b, c0_root, c0_bias,
           c1_l1_w, c1_l1_b, c1_l2_w, c1_l2_b, c1_root, c1_bias,
           c2_l1_w, c2_l1_b, c2_l2_w, c2_l2_b, c2_root, c2_bias,
           fc1_w, fc1_b, fc2_w, fc2_b, fc3_w, fc3_b):
    f32 = jnp.float32
    i32 = jnp.int32

    x_col = jnp.pad(x.astype(i32), (0, N_PAD - N_NODES)).reshape(N_PAD, 1)
    src4d = jnp.pad(edge_index[0].astype(i32),
                    (0, E_PAD - N_EDGES)).reshape(NW, KJ, 128)
    dst4d = jnp.pad(edge_index[1].astype(i32), (0, E_PAD - N_EDGES),
                    constant_values=N_NODES).reshape(NW, KJ, 128)
    ea = jnp.pad(edge_attr.astype(f32), ((0, E_PAD - N_EDGES), (0, 0)))
    batch_row = jnp.pad(batch.astype(i32), (0, N_PAD - N_NODES),
                        constant_values=127).reshape(1, N_PAD)
    emb_pad = jnp.pad(emb_table.astype(f32), ((0, 3), (0, 0)))

    layers = [
        (c0_l1_w, c0_l1_b, c0_l2_w, c0_l2_b, c0_root, c0_bias),
        (c1_l1_w, c1_l1_b, c1_l2_w, c1_l2_b, c1_root, c1_bias),
        (c2_l1_w, c2_l1_b, c2_l2_w, c2_l2_b, c2_root, c2_bias),
    ]

    h = _embed(x_col, emb_pad)

    for li, (mi, mi_p, mo) in enumerate(DIMS_K):
        l1w, l1b, l2w, l2b, root, bias = layers[li]
        l2bm = jnp.pad(l2b.reshape(mi, mo), ((0, mi_p - mi), (0, 0)))
        root_p = jnp.pad(root, ((0, mi_p - mi), (0, 0)))
        bias2 = bias.reshape(1, mo)
        zeros_nd = jnp.zeros((N_PAD, mo), f32)

        hsrc = _gather_rows(h, src4d, mi_p)
        msg = _msg_call(ea, hsrc, l1w.T, l1b.reshape(128, 1), l2w.T,
                        l2bm.T, mi, mo, mi_p)
        aggp = _scatter_add(msg, dst4d, zeros_nd, mo)
        if li < 2:
            h = _update_call(h, aggp[0], aggp[1], root_p, bias2, mi_p, mo)
        else:
            return _update_readout_call(
                h, aggp[0], aggp[1], root_p, bias2, batch_row,
                fc1_w, fc1_b.reshape(1, 32),
                fc2_w, fc2_b.reshape(1, 16),
                fc3_w, fc3_b.reshape(1, 1), mi_p, mo)
